# baseline (speedup 1.0000x reference)
import functools
import math

import jax
import jax.numpy as jnp
from jax import lax
from jax.experimental import pallas as pl
from jax.experimental.pallas import tpu as pltpu


def _prep_kernel(ce_ref, ctx_ref, wqk_ref, wctx_ref, bias_ref, aq_ref, ak_ref, *, C):
    y = jnp.dot(ce_ref[...], wqk_ref[...], preferred_element_type=jnp.float32)
    y = y + jnp.dot(ctx_ref[...], wctx_ref[...],
                    preferred_element_type=jnp.float32)
    y = y + bias_ref[...]
    aq_ref[...] = y[:, :C]
    ak_ref[...] = y[:, C:]


def _prep_terms(coords_enc, ctx_t, wqk, wctx, bias, *, C, tiles=2):
    plane = coords_enc.shape[0]
    C_qk = coords_enc.shape[1]
    tm = plane // tiles
    return pl.pallas_call(
        functools.partial(_prep_kernel, C=C),
        out_shape=(jax.ShapeDtypeStruct((plane, C), jnp.float32),
                   jax.ShapeDtypeStruct((plane, C), jnp.float32)),
        grid_spec=pltpu.PrefetchScalarGridSpec(
            num_scalar_prefetch=0,
            grid=(tiles,),
            in_specs=[
                pl.BlockSpec((tm, C_qk), lambda i: (i, 0)),
                pl.BlockSpec((tm, ctx_t.shape[1]), lambda i: (i, 0)),
                pl.BlockSpec((C_qk, 2 * C), lambda i: (0, 0)),
                pl.BlockSpec((wctx.shape[0], 2 * C), lambda i: (0, 0)),
                pl.BlockSpec((1, 2 * C), lambda i: (0, 0)),
            ],
            out_specs=[pl.BlockSpec((tm, C), lambda i: (i, 0)),
                       pl.BlockSpec((tm, C), lambda i: (i, 0))],
        ),
        compiler_params=pltpu.CompilerParams(dimension_semantics=("parallel",)),
    )(coords_enc, ctx_t, wqk, wctx, bias)


def _attn_kernel(tbl_ref, xq_ref, xkv_ref, aq_ref, ak_ref, wq_ref, wkv_ref,
                 vb_ref, pwx_ref, pb_ref, o_ref, *, ws, Wp, C, nh, bn):
    b = pl.program_id(0)
    r = pl.program_id(1)
    g = b // bn
    hd = C // nh
    nwx = Wp // ws
    rows = ws * Wp
    L = ws * ws
    Lp = ((L + 7) // 8) * 8

    kvr = tbl_ref[b, r]

    xq = xq_ref[0].reshape(rows, C)
    xkv = xkv_ref[0].reshape(rows, C)

    q = jnp.dot(xq, wq_ref[...], preferred_element_type=jnp.float32)
    q = q + aq_ref[pl.ds(r * rows, rows), :]
    kv = jnp.dot(xkv, wkv_ref[...], preferred_element_type=jnp.float32)
    k = (kv[:, :C] + ak_ref[pl.ds(kvr * rows, rows), :]).reshape(ws, Wp, C)
    v = (kv[:, C:] + vb_ref[...]).reshape(ws, Wp, C)

    if nwx > 1:
        k_l = jnp.concatenate([k[:, ws:, :], k[:, Wp - ws:, :]], axis=1)
        k_r = jnp.concatenate([k[:, :ws, :], k[:, :Wp - ws, :]], axis=1)
        v_l = jnp.concatenate([v[:, ws:, :], v[:, Wp - ws:, :]], axis=1)
        v_r = jnp.concatenate([v[:, :ws, :], v[:, :Wp - ws, :]], axis=1)
        k = jnp.where(g == 2, k_l, jnp.where(g == 3, k_r, k))
        v = jnp.where(g == 2, v_l, jnp.where(g == 3, v_r, v))

    q = q.reshape(ws, Wp, C)

    rid = lax.broadcasted_iota(jnp.int32, (nh * Lp, C), 0) // Lp
    cid = lax.broadcasted_iota(jnp.int32, (nh * Lp, C), 1) // hd
    mask = rid == cid

    outs = []
    for wx in range(nwx):
        cs = slice(wx * ws, (wx + 1) * ws)
        qw = q[:, cs, :].reshape(L, C)
        kw = k[:, cs, :].reshape(L, C)
        vw = v[:, cs, :].reshape(L, C)
        qp = jnp.concatenate([qw, jnp.zeros((Lp - L, C), jnp.float32)], axis=0)
        qrep = jnp.where(mask, jnp.tile(qp, (nh, 1)), 0.0)
        s = lax.dot_general(qrep, kw, (((1,), (1,)), ((), ())),
                            preferred_element_type=jnp.float32)
        p = jnp.exp(jnp.minimum(s, 60.0))
        p = p * pl.reciprocal(jnp.sum(p, axis=-1, keepdims=True))
        pv = jnp.dot(p, vw, preferred_element_type=jnp.float32)
        pv2 = jnp.concatenate([pv[h * Lp:(h + 1) * Lp] for h in range(nh)],
                              axis=1)
        ow = jnp.dot(pv2, pwx_ref[...],
                     preferred_element_type=jnp.float32)[:L]
        outs.append(ow.reshape(ws, ws, C))

    o_row = jnp.concatenate(outs, axis=1).reshape(rows, C)
    res = o_row + pb_ref[...]
    o_ref[0] = res.reshape(ws, Wp, C)


def _fused_attention(x4, kv_row, add_q, add_k, w_q, w_kv, v_b, proj_w, proj_b,
                     *, ws, nh, bn):
    B, Hp, Wp, C = x4.shape
    _h = Hp // ws
    plane = Hp * Wp

    q_map = lambda b, r, tbl: (b, r, 0, 0)
    kv_map = lambda b, r, tbl: (b, tbl[b, r], 0, 0)
    res_map = lambda b, r, tbl: (0, 0)

    out = pl.pallas_call(
        functools.partial(_attn_kernel, ws=ws, Wp=Wp, C=C, nh=nh, bn=bn),
        out_shape=jax.ShapeDtypeStruct((B, Hp, Wp, C), jnp.float32),
        grid_spec=pltpu.PrefetchScalarGridSpec(
            num_scalar_prefetch=1,
            grid=(B, _h),
            in_specs=[
                pl.BlockSpec((1, ws, Wp, C), q_map),
                pl.BlockSpec((1, ws, Wp, C), kv_map),
                pl.BlockSpec((plane, C), res_map),
                pl.BlockSpec((plane, C), res_map),
                pl.BlockSpec((C, C), res_map),
                pl.BlockSpec((C, 2 * C), res_map),
                pl.BlockSpec((1, C), res_map),
                pl.BlockSpec((proj_w.shape[0], C), res_map),
                pl.BlockSpec((1, C), res_map),
            ],
            out_specs=pl.BlockSpec((1, ws, Wp, C), q_map),
        ),
        compiler_params=pltpu.CompilerParams(
            dimension_semantics=("parallel", "parallel")),
    )(kv_row, x4, x4, add_q, add_k, w_q, w_kv, v_b, proj_w, proj_b)
    return out


def _sine_pos_enc(Hp, Wp, dim):
    freqs = jnp.arange(dim // 4, dtype=jnp.float32)
    yy, xx = jnp.meshgrid(jnp.arange(Hp, dtype=jnp.float32),
                          jnp.arange(Wp, dtype=jnp.float32), indexing="ij")
    wx = 3.14 * xx[..., None] * freqs * (1.0 / 200.0)
    wy = 3.14 * yy[..., None] * freqs * (1.0 / 200.0)
    return jnp.concatenate([jnp.sin(wx), jnp.cos(wx), jnp.sin(wy), jnp.cos(wy)],
                           axis=-1).reshape(Hp * Wp, dim)


def kernel(x, context, ctx_w, ctx_b, q_w, q_b, k_w, k_b, v_w, v_b,
           proj_w, proj_b):
    B, N, C = x.shape
    H = W = int(math.isqrt(N))
    ws = 7
    nh = 8
    vert_c_dim = q_w.shape[0] - C
    C_qk = C + vert_c_dim
    hd = C // nh
    scale = hd ** (-0.5)
    bn = B // 5
    _h = H // ws

    q_w = q_w * scale
    q_b = q_b * scale

    wqk = jnp.concatenate([q_w, k_w], axis=1)
    w_ctx2qk = ctx_w @ wqk[C:]
    bias = (jnp.concatenate([q_b, k_b], axis=0) + ctx_b @ wqk[C:]).reshape(1, 2 * C)

    coords_enc = _sine_pos_enc(H, W, C_qk)
    ctx_t = context.reshape(context.shape[1], H * W).T

    add_q, add_k = _prep_terms(coords_enc, ctx_t, wqk, w_ctx2qk, bias, C=C)

    wy = jnp.arange(_h, dtype=jnp.int32)
    row_up = jnp.minimum(wy + 1, _h - 1)
    row_dn = jnp.maximum(wy - 1, 0)
    gvec = (jnp.arange(B, dtype=jnp.int32) // bn)[:, None]
    kv_row = jnp.where(gvec == 0, row_up[None, :],
                       jnp.where(gvec == 1, row_dn[None, :],
                                 jnp.broadcast_to(wy[None, :], (B, _h)))).astype(jnp.int32)

    x4 = x.reshape(B, H, W, C)
    w_kv = jnp.concatenate([k_w[:C], v_w], axis=1)

    hid = jnp.arange(nh * C, dtype=jnp.int32)
    keep = ((hid % C) // hd) == (hid // C)
    pw_exp = jnp.where(keep[:, None], jnp.tile(proj_w, (nh, 1)), 0.0)

    out = _fused_attention(x4, kv_row, add_q, add_k, q_w[:C], w_kv,
                           v_b.reshape(1, C), pw_exp, proj_b.reshape(1, C),
                           ws=ws, nh=nh, bn=bn)
    return out.reshape(B, N, C)


# speedup vs baseline: 1.9485x; 1.9485x over previous
import functools
import math

import jax
import jax.numpy as jnp
from jax import lax
from jax.experimental import pallas as pl
from jax.experimental.pallas import tpu as pltpu


def _prep_kernel(ce_ref, ctx_ref, wqk_ref, wctx_ref, bias_ref, aq_ref, ak_ref, *, C):
    y = jnp.dot(ce_ref[...], wqk_ref[...], preferred_element_type=jnp.float32)
    y = y + jnp.dot(ctx_ref[...], wctx_ref[...],
                    preferred_element_type=jnp.float32)
    y = y + bias_ref[...]
    aq_ref[...] = y[:, :C]
    ak_ref[...] = y[:, C:]


def _prep_terms(coords_enc, ctx_t, wqk, wctx, bias, *, C, tiles=2):
    plane = coords_enc.shape[0]
    C_qk = coords_enc.shape[1]
    tm = plane // tiles
    return pl.pallas_call(
        functools.partial(_prep_kernel, C=C),
        out_shape=(jax.ShapeDtypeStruct((plane, C), jnp.float32),
                   jax.ShapeDtypeStruct((plane, C), jnp.float32)),
        grid_spec=pltpu.PrefetchScalarGridSpec(
            num_scalar_prefetch=0,
            grid=(tiles,),
            in_specs=[
                pl.BlockSpec((tm, C_qk), lambda i: (i, 0)),
                pl.BlockSpec((tm, ctx_t.shape[1]), lambda i: (i, 0)),
                pl.BlockSpec((C_qk, 2 * C), lambda i: (0, 0)),
                pl.BlockSpec((wctx.shape[0], 2 * C), lambda i: (0, 0)),
                pl.BlockSpec((1, 2 * C), lambda i: (0, 0)),
            ],
            out_specs=[pl.BlockSpec((tm, C), lambda i: (i, 0)),
                       pl.BlockSpec((tm, C), lambda i: (i, 0))],
        ),
        compiler_params=pltpu.CompilerParams(dimension_semantics=("parallel",)),
    )(coords_enc, ctx_t, wqk, wctx, bias)


def _attn_kernel(tbl_ref, xq_ref, xkv_ref, aq_ref, ak_ref, wq_ref, wkv_ref,
                 vb_ref, pwx_ref, pb_ref, o_ref, *, ws, Wp, C, nh, bn):
    b = pl.program_id(0)
    r = pl.program_id(1)
    g = b // bn
    hd = C // nh
    nwx = Wp // ws
    rows = ws * Wp
    L = ws * ws
    Lp = ((L + 7) // 8) * 8

    kvr = tbl_ref[b, r]

    xq = xq_ref[0].reshape(rows, C)
    xkv = xkv_ref[0].reshape(rows, C)

    q = jnp.dot(xq, wq_ref[...], preferred_element_type=jnp.float32)
    q = q + aq_ref[pl.ds(r * rows, rows), :]
    kv = jnp.dot(xkv, wkv_ref[...], preferred_element_type=jnp.float32)
    k = (kv[:, :C] + ak_ref[pl.ds(kvr * rows, rows), :]).reshape(ws, Wp, C)
    v = (kv[:, C:] + vb_ref[...]).reshape(ws, Wp, C)

    if nwx > 1:
        k_l = jnp.concatenate([k[:, ws:, :], k[:, Wp - ws:, :]], axis=1)
        k_r = jnp.concatenate([k[:, :ws, :], k[:, :Wp - ws, :]], axis=1)
        v_l = jnp.concatenate([v[:, ws:, :], v[:, Wp - ws:, :]], axis=1)
        v_r = jnp.concatenate([v[:, :ws, :], v[:, :Wp - ws, :]], axis=1)
        k = jnp.where(g == 2, k_l, jnp.where(g == 3, k_r, k))
        v = jnp.where(g == 2, v_l, jnp.where(g == 3, v_r, v))

    q = q.reshape(ws, Wp, C)

    rid = lax.broadcasted_iota(jnp.int32, (nh * Lp, C), 0) // Lp
    cid = lax.broadcasted_iota(jnp.int32, (nh * Lp, C), 1) // hd
    mask = rid == cid

    outs = []
    for wx in range(nwx):
        cs = slice(wx * ws, (wx + 1) * ws)
        qw = q[:, cs, :].reshape(L, C)
        kw = k[:, cs, :].reshape(L, C)
        vw = v[:, cs, :].reshape(L, C)
        qp = jnp.concatenate([qw, jnp.zeros((Lp - L, C), jnp.float32)], axis=0)
        qrep = jnp.where(mask, jnp.tile(qp, (nh, 1)), 0.0)
        s = lax.dot_general(qrep, kw, (((1,), (1,)), ((), ())),
                            preferred_element_type=jnp.float32)
        p = jnp.exp(jnp.minimum(s, 60.0))
        p = p * pl.reciprocal(jnp.sum(p, axis=-1, keepdims=True))
        pv = jnp.dot(p, vw, preferred_element_type=jnp.float32)
        pv = jnp.where(mask, pv, 0.0)
        ow = pv.reshape(nh, Lp, C).sum(axis=0)[:L]
        outs.append(ow.reshape(ws, ws, C))

    o_row = jnp.concatenate(outs, axis=1).reshape(rows, C)
    res = jnp.dot(o_row, pwx_ref[...], preferred_element_type=jnp.float32)
    res = res + pb_ref[...]
    o_ref[0] = res.reshape(ws, Wp, C)


def _fused_attention(x4, kv_row, add_q, add_k, w_q, w_kv, v_b, proj_w, proj_b,
                     *, ws, nh, bn):
    B, Hp, Wp, C = x4.shape
    _h = Hp // ws
    plane = Hp * Wp

    q_map = lambda b, r, tbl: (b, r, 0, 0)
    kv_map = lambda b, r, tbl: (b, tbl[b, r], 0, 0)
    res_map = lambda b, r, tbl: (0, 0)

    out = pl.pallas_call(
        functools.partial(_attn_kernel, ws=ws, Wp=Wp, C=C, nh=nh, bn=bn),
        out_shape=jax.ShapeDtypeStruct((B, Hp, Wp, C), jnp.float32),
        grid_spec=pltpu.PrefetchScalarGridSpec(
            num_scalar_prefetch=1,
            grid=(B, _h),
            in_specs=[
                pl.BlockSpec((1, ws, Wp, C), q_map),
                pl.BlockSpec((1, ws, Wp, C), kv_map),
                pl.BlockSpec((plane, C), res_map),
                pl.BlockSpec((plane, C), res_map),
                pl.BlockSpec((C, C), res_map),
                pl.BlockSpec((C, 2 * C), res_map),
                pl.BlockSpec((1, C), res_map),
                pl.BlockSpec((C, C), res_map),
                pl.BlockSpec((1, C), res_map),
            ],
            out_specs=pl.BlockSpec((1, ws, Wp, C), q_map),
        ),
        compiler_params=pltpu.CompilerParams(
            dimension_semantics=("parallel", "parallel")),
    )(kv_row, x4, x4, add_q, add_k, w_q, w_kv, v_b, proj_w, proj_b)
    return out


def _sine_pos_enc(Hp, Wp, dim):
    freqs = jnp.arange(dim // 4, dtype=jnp.float32)
    yy, xx = jnp.meshgrid(jnp.arange(Hp, dtype=jnp.float32),
                          jnp.arange(Wp, dtype=jnp.float32), indexing="ij")
    wx = 3.14 * xx[..., None] * freqs * (1.0 / 200.0)
    wy = 3.14 * yy[..., None] * freqs * (1.0 / 200.0)
    return jnp.concatenate([jnp.sin(wx), jnp.cos(wx), jnp.sin(wy), jnp.cos(wy)],
                           axis=-1).reshape(Hp * Wp, dim)


def kernel(x, context, ctx_w, ctx_b, q_w, q_b, k_w, k_b, v_w, v_b,
           proj_w, proj_b):
    B, N, C = x.shape
    H = W = int(math.isqrt(N))
    ws = 7
    nh = 8
    vert_c_dim = q_w.shape[0] - C
    C_qk = C + vert_c_dim
    hd = C // nh
    scale = hd ** (-0.5)
    bn = B // 5
    _h = H // ws

    q_w = q_w * scale
    q_b = q_b * scale

    wqk = jnp.concatenate([q_w, k_w], axis=1)
    w_ctx2qk = ctx_w @ wqk[C:]
    bias = (jnp.concatenate([q_b, k_b], axis=0) + ctx_b @ wqk[C:]).reshape(1, 2 * C)

    coords_enc = _sine_pos_enc(H, W, C_qk)
    ctx_t = context.reshape(context.shape[1], H * W).T

    add_q, add_k = _prep_terms(coords_enc, ctx_t, wqk, w_ctx2qk, bias, C=C)

    wy = jnp.arange(_h, dtype=jnp.int32)
    row_up = jnp.minimum(wy + 1, _h - 1)
    row_dn = jnp.maximum(wy - 1, 0)
    gvec = (jnp.arange(B, dtype=jnp.int32) // bn)[:, None]
    kv_row = jnp.where(gvec == 0, row_up[None, :],
                       jnp.where(gvec == 1, row_dn[None, :],
                                 jnp.broadcast_to(wy[None, :], (B, _h)))).astype(jnp.int32)

    x4 = x.reshape(B, H, W, C)
    w_kv = jnp.concatenate([k_w[:C], v_w], axis=1)

    out = _fused_attention(x4, kv_row, add_q, add_k, q_w[:C], w_kv,
                           v_b.reshape(1, C), proj_w, proj_b.reshape(1, C),
                           ws=ws, nh=nh, bn=bn)
    return out.reshape(B, N, C)


# speedup vs baseline: 2.0623x; 1.0584x over previous
import functools
import math

import jax
import jax.numpy as jnp
from jax import lax
from jax.experimental import pallas as pl
from jax.experimental.pallas import tpu as pltpu


def _prep_kernel(ce_ref, ctx_ref, wqk_ref, wctx_ref, bias_ref, aq_ref, ak_ref,
                 *, C, _h, nwx, ws):
    y = jnp.dot(ce_ref[...], wqk_ref[...], preferred_element_type=jnp.float32)
    y2 = lax.dot_general(ctx_ref[...], wctx_ref[...],
                         (((0,), (0,)), ((), ())),
                         preferred_element_type=jnp.float32)
    y2 = y2.reshape(_h, ws, nwx, ws, 2 * C).transpose(0, 2, 1, 3, 4)
    y2 = jnp.pad(y2, ((0, 0), (0, 0), (0, 0), (0, 8 - ws), (0, 0)))
    y2 = y2.reshape(_h * nwx * ws * 8, 2 * C)
    y = y + y2 + bias_ref[...]
    aq_ref[...] = y[:, :C]
    ak_ref[...] = y[:, C:]


def _prep_terms(ce_wm, ctx_raw, wqk, wctx, bias, *, C, _h, nwx, ws):
    planep = ce_wm.shape[0]
    C_qk = ce_wm.shape[1]
    return pl.pallas_call(
        functools.partial(_prep_kernel, C=C, _h=_h, nwx=nwx, ws=ws),
        out_shape=(jax.ShapeDtypeStruct((planep, C), jnp.float32),
                   jax.ShapeDtypeStruct((planep, C), jnp.float32)),
        grid_spec=pltpu.PrefetchScalarGridSpec(
            num_scalar_prefetch=0,
            grid=(1,),
            in_specs=[
                pl.BlockSpec((planep, C_qk), lambda i: (0, 0)),
                pl.BlockSpec(ctx_raw.shape, lambda i: (0, 0)),
                pl.BlockSpec((C_qk, 2 * C), lambda i: (0, 0)),
                pl.BlockSpec((wctx.shape[0], 2 * C), lambda i: (0, 0)),
                pl.BlockSpec((1, 2 * C), lambda i: (0, 0)),
            ],
            out_specs=[pl.BlockSpec((planep, C), lambda i: (0, 0)),
                       pl.BlockSpec((planep, C), lambda i: (0, 0))],
        ),
        compiler_params=pltpu.CompilerParams(dimension_semantics=("arbitrary",)),
    )(ce_wm, ctx_raw, wqk, wctx, bias)


def _assemble(ref6, ws, nwx, C):
    a = ref6[0, 0]
    a = jnp.transpose(a, (1, 0, 2, 3))
    a = jnp.pad(a, ((0, 0), (0, 0), (0, 8 - ws), (0, 0)))
    return a.reshape(nwx * ws * 8, C)


def _attn_kernel(tbl_ref, xq_ref, xkv_ref, aq_ref, ak_ref, wq_ref, wkv_ref,
                 vb_ref, pw_ref, pb_ref, o_ref, k_ref, v_ref,
                 *, ws, nwx, C, nh, bn):
    b = pl.program_id(0)
    r = pl.program_id(1)
    g = b // bn
    hd = C // nh
    Lp = ws * 8
    rows = nwx * Lp
    kvr = tbl_ref[b, r]

    xq = _assemble(xq_ref, ws, nwx, C)
    xkv = _assemble(xkv_ref, ws, nwx, C)

    q = jnp.dot(xq, wq_ref[...], preferred_element_type=jnp.float32)
    q = q + aq_ref[pl.ds(r * rows, rows), :]
    kv = jnp.dot(xkv, wkv_ref[...], preferred_element_type=jnp.float32)
    k_ref[...] = kv[:, :C] + ak_ref[pl.ds(kvr * rows, rows), :]
    v_ref[...] = kv[:, C:] + vb_ref[...]

    rid = lax.broadcasted_iota(jnp.int32, (nh * Lp, C), 0) // Lp
    cid = lax.broadcasted_iota(jnp.int32, (nh * Lp, C), 1) // hd
    mask = rid == cid

    lane = lax.broadcasted_iota(jnp.int32, (1, Lp), 1)
    bound = jnp.where(lane % 8 < ws, 60.0, -1e30)

    outs = []
    for wx in range(nwx):
        wl = min(wx + 1, nwx - 1)
        wr = max(wx - 1, 0)
        kvw = jnp.where(g == 2, wl, jnp.where(g == 3, wr, wx))
        qw = q[wx * Lp:(wx + 1) * Lp]
        kw = k_ref[pl.ds(kvw * Lp, Lp), :]
        vw = v_ref[pl.ds(kvw * Lp, Lp), :]
        qrep = jnp.where(mask, jnp.tile(qw, (nh, 1)), 0.0)
        s = lax.dot_general(qrep, kw, (((1,), (1,)), ((), ())),
                            preferred_element_type=jnp.float32)
        p = jnp.exp(jnp.minimum(s, bound))
        p = p * pl.reciprocal(jnp.sum(p, axis=-1, keepdims=True))
        pv = jnp.dot(p, vw, preferred_element_type=jnp.float32)
        pv = jnp.where(mask, pv, 0.0)
        outs.append(pv.reshape(nh, Lp, C).sum(axis=0))

    o_all = jnp.concatenate(outs, axis=0)
    res = jnp.dot(o_all, pw_ref[...], preferred_element_type=jnp.float32)
    res = res + pb_ref[...]
    res = res.reshape(nwx, ws, 8, C)[:, :, :ws, :]
    o_ref[0, 0] = jnp.transpose(res, (1, 0, 2, 3))


def _fused_attention(x6, kv_row, add_q, add_k, w_q, w_kv, v_b, proj_w, proj_b,
                     *, ws, nwx, nh, bn):
    B, _h = x6.shape[0], x6.shape[1]
    C = x6.shape[5]
    rows = nwx * ws * 8

    q_map = lambda b, r, tbl: (b, r, 0, 0, 0, 0)
    kv_map = lambda b, r, tbl: (b, tbl[b, r], 0, 0, 0, 0)
    res_map = lambda b, r, tbl: (0, 0)

    out = pl.pallas_call(
        functools.partial(_attn_kernel, ws=ws, nwx=nwx, C=C, nh=nh, bn=bn),
        out_shape=jax.ShapeDtypeStruct((B, _h, ws, nwx, ws, C), jnp.float32),
        grid_spec=pltpu.PrefetchScalarGridSpec(
            num_scalar_prefetch=1,
            grid=(B, _h),
            in_specs=[
                pl.BlockSpec((1, 1, ws, nwx, ws, C), q_map),
                pl.BlockSpec((1, 1, ws, nwx, ws, C), kv_map),
                pl.BlockSpec((_h * rows, C), res_map),
                pl.BlockSpec((_h * rows, C), res_map),
                pl.BlockSpec((C, C), res_map),
                pl.BlockSpec((C, 2 * C), res_map),
                pl.BlockSpec((1, C), res_map),
                pl.BlockSpec((C, C), res_map),
                pl.BlockSpec((1, C), res_map),
            ],
            out_specs=pl.BlockSpec((1, 1, ws, nwx, ws, C), q_map),
            scratch_shapes=[pltpu.VMEM((rows, C), jnp.float32),
                            pltpu.VMEM((rows, C), jnp.float32)],
        ),
        compiler_params=pltpu.CompilerParams(
            dimension_semantics=("parallel", "parallel")),
    )(kv_row, x6, x6, add_q, add_k, w_q, w_kv, v_b, proj_w, proj_b)
    return out


def _sine_pos_enc(coords, dim):
    freqs = jnp.arange(dim // 4, dtype=jnp.float32)
    wx = 3.14 * coords[..., 0:1] * freqs * (1.0 / 200.0)
    wy = 3.14 * coords[..., 1:2] * freqs * (1.0 / 200.0)
    return jnp.concatenate([jnp.sin(wx), jnp.cos(wx), jnp.sin(wy), jnp.cos(wy)],
                           axis=-1)


def kernel(x, context, ctx_w, ctx_b, q_w, q_b, k_w, k_b, v_w, v_b,
           proj_w, proj_b):
    B, N, C = x.shape
    H = W = int(math.isqrt(N))
    ws = 7
    nh = 8
    vert_c_dim = q_w.shape[0] - C
    C_qk = C + vert_c_dim
    hd = C // nh
    scale = hd ** (-0.5)
    bn = B // 5
    _h, nwx = H // ws, W // ws
    planep = _h * nwx * ws * 8

    q_w = q_w * scale
    q_b = q_b * scale

    wqk = jnp.concatenate([q_w, k_w], axis=1)
    w_ctx2qk = ctx_w @ wqk[C:]
    bias = (jnp.concatenate([q_b, k_b], axis=0) + ctx_b @ wqk[C:]).reshape(1, 2 * C)

    rr, wxs, yy, xx = jnp.meshgrid(
        jnp.arange(_h, dtype=jnp.float32), jnp.arange(nwx, dtype=jnp.float32),
        jnp.arange(ws, dtype=jnp.float32), jnp.arange(8, dtype=jnp.float32),
        indexing="ij")
    cx = (wxs * ws + jnp.minimum(xx, ws - 1.0)).reshape(-1)
    cy = (rr * ws + yy).reshape(-1)
    coords_wm = jnp.stack([cx, cy], axis=-1)
    ce_wm = _sine_pos_enc(coords_wm, C_qk)

    ctx_raw = context.reshape(context.shape[1], H * W)

    add_q, add_k = _prep_terms(ce_wm, ctx_raw, wqk, w_ctx2qk, bias,
                               C=C, _h=_h, nwx=nwx, ws=ws)

    wy = jnp.arange(_h, dtype=jnp.int32)
    row_up = jnp.minimum(wy + 1, _h - 1)
    row_dn = jnp.maximum(wy - 1, 0)
    gvec = (jnp.arange(B, dtype=jnp.int32) // bn)[:, None]
    kv_row = jnp.where(gvec == 0, row_up[None, :],
                       jnp.where(gvec == 1, row_dn[None, :],
                                 jnp.broadcast_to(wy[None, :], (B, _h)))).astype(jnp.int32)

    x6 = x.reshape(B, _h, ws, nwx, ws, C)
    w_kv = jnp.concatenate([k_w[:C], v_w], axis=1)

    out = _fused_attention(x6, kv_row, add_q, add_k, q_w[:C], w_kv,
                           v_b.reshape(1, C), proj_w, proj_b.reshape(1, C),
                           ws=ws, nwx=nwx, nh=nh, bn=bn)
    return out.reshape(B, N, C)


# speedup vs baseline: 2.6375x; 1.2789x over previous
import functools
import math

import jax
import jax.numpy as jnp
from jax import lax
from jax.experimental import pallas as pl
from jax.experimental.pallas import tpu as pltpu


def _prep_kernel(ce_ref, ctx_ref, wqk_ref, wctx_ref, bias_ref, aq_ref, ak_ref,
                 *, C, _h, nwx, ws):
    y = jnp.dot(ce_ref[...], wqk_ref[...], preferred_element_type=jnp.float32)
    y2 = lax.dot_general(ctx_ref[...], wctx_ref[...],
                         (((0,), (0,)), ((), ())),
                         preferred_element_type=jnp.float32)
    y2 = y2.reshape(_h, ws, nwx, ws, 2 * C).transpose(0, 2, 1, 3, 4)
    y2 = jnp.pad(y2, ((0, 0), (0, 0), (0, 0), (0, 8 - ws), (0, 0)))
    y2 = y2.reshape(_h * nwx * ws * 8, 2 * C)
    y = y + y2 + bias_ref[...]
    aq_ref[...] = y[:, :C]
    ak_ref[...] = y[:, C:]


def _prep_terms(ce_wm, ctx_raw, wqk, wctx, bias, *, C, _h, nwx, ws):
    planep = ce_wm.shape[0]
    C_qk = ce_wm.shape[1]
    return pl.pallas_call(
        functools.partial(_prep_kernel, C=C, _h=_h, nwx=nwx, ws=ws),
        out_shape=(jax.ShapeDtypeStruct((planep, C), jnp.float32),
                   jax.ShapeDtypeStruct((planep, C), jnp.float32)),
        grid_spec=pltpu.PrefetchScalarGridSpec(
            num_scalar_prefetch=0,
            grid=(1,),
            in_specs=[
                pl.BlockSpec((planep, C_qk), lambda i: (0, 0)),
                pl.BlockSpec(ctx_raw.shape, lambda i: (0, 0)),
                pl.BlockSpec((C_qk, 2 * C), lambda i: (0, 0)),
                pl.BlockSpec((wctx.shape[0], 2 * C), lambda i: (0, 0)),
                pl.BlockSpec((1, 2 * C), lambda i: (0, 0)),
            ],
            out_specs=[pl.BlockSpec((planep, C), lambda i: (0, 0)),
                       pl.BlockSpec((planep, C), lambda i: (0, 0))],
        ),
        compiler_params=pltpu.CompilerParams(dimension_semantics=("arbitrary",)),
    )(ce_wm, ctx_raw, wqk, wctx, bias)


def _attn_kernel(x_ref, aq_ref, ak_ref, wq_ref, wkv_ref,
                 vb_ref, pw_ref, pb_ref, o_ref, k_ref, v_ref,
                 *, ws, nwx, _h, C, nh, bn):
    b = pl.program_id(0)
    g = b // bn
    hd = C // nh
    Lp = ws * 8
    rows = _h * nwx * Lp

    a = x_ref[0]
    a = jnp.transpose(a, (0, 2, 1, 3, 4))
    a = jnp.pad(a, ((0, 0), (0, 0), (0, 0), (0, 8 - ws), (0, 0)))
    xw = a.reshape(rows, C)

    q = jnp.dot(xw, wq_ref[...], preferred_element_type=jnp.float32)
    q = q + aq_ref[...]
    kv = jnp.dot(xw, wkv_ref[...], preferred_element_type=jnp.float32)
    k_ref[...] = kv[:, :C] + ak_ref[...]
    v_ref[...] = kv[:, C:] + vb_ref[...]

    rid = lax.broadcasted_iota(jnp.int32, (nh * Lp, C), 0) // Lp
    cid = lax.broadcasted_iota(jnp.int32, (nh * Lp, C), 1) // hd
    mask = rid == cid

    lane = lax.broadcasted_iota(jnp.int32, (1, Lp), 1)
    bound = jnp.where(lane % 8 < ws, 60.0, -1e30)

    outs = []
    for r in range(_h):
        ru = min(r + 1, _h - 1)
        rd = max(r - 1, 0)
        kvr = jnp.where(g == 0, ru, jnp.where(g == 1, rd, r))
        for wx in range(nwx):
            wl = min(wx + 1, nwx - 1)
            wr = max(wx - 1, 0)
            kvw = jnp.where(g == 2, wl, jnp.where(g == 3, wr, wx))
            slab = (kvr * nwx + kvw) * Lp
            qw = q[(r * nwx + wx) * Lp:(r * nwx + wx + 1) * Lp]
            kw = k_ref[pl.ds(slab, Lp), :]
            vw = v_ref[pl.ds(slab, Lp), :]
            qrep = jnp.where(mask, jnp.tile(qw, (nh, 1)), 0.0)
            s = lax.dot_general(qrep, kw, (((1,), (1,)), ((), ())),
                                preferred_element_type=jnp.float32)
            p = jnp.exp(jnp.minimum(s, bound))
            p = p * pl.reciprocal(jnp.sum(p, axis=-1, keepdims=True))
            pv = jnp.dot(p, vw, preferred_element_type=jnp.float32)
            pv = jnp.where(mask, pv, 0.0)
            outs.append(pv.reshape(nh, Lp, C).sum(axis=0))

    o_all = jnp.concatenate(outs, axis=0)
    res = jnp.dot(o_all, pw_ref[...], preferred_element_type=jnp.float32)
    res = res + pb_ref[...]
    res = res.reshape(_h, nwx, ws, 8, C)[:, :, :, :ws, :]
    o_ref[0] = jnp.transpose(res, (0, 2, 1, 3, 4))


def _fused_attention(x6, add_q, add_k, w_q, w_kv, v_b, proj_w, proj_b,
                     *, ws, nwx, nh, bn):
    B, _h = x6.shape[0], x6.shape[1]
    C = x6.shape[5]
    rows = _h * nwx * ws * 8

    b_map = lambda b: (b, 0, 0, 0, 0, 0)
    res_map = lambda b: (0, 0)

    out = pl.pallas_call(
        functools.partial(_attn_kernel, ws=ws, nwx=nwx, _h=_h, C=C, nh=nh,
                          bn=bn),
        out_shape=jax.ShapeDtypeStruct((B, _h, ws, nwx, ws, C), jnp.float32),
        grid_spec=pltpu.PrefetchScalarGridSpec(
            num_scalar_prefetch=0,
            grid=(B,),
            in_specs=[
                pl.BlockSpec((1, _h, ws, nwx, ws, C), b_map),
                pl.BlockSpec((rows, C), res_map),
                pl.BlockSpec((rows, C), res_map),
                pl.BlockSpec((C, C), res_map),
                pl.BlockSpec((C, 2 * C), res_map),
                pl.BlockSpec((1, C), res_map),
                pl.BlockSpec((C, C), res_map),
                pl.BlockSpec((1, C), res_map),
            ],
            out_specs=pl.BlockSpec((1, _h, ws, nwx, ws, C), b_map),
            scratch_shapes=[pltpu.VMEM((rows, C), jnp.float32),
                            pltpu.VMEM((rows, C), jnp.float32)],
        ),
        compiler_params=pltpu.CompilerParams(
            dimension_semantics=("parallel",)),
    )(x6, add_q, add_k, w_q, w_kv, v_b, proj_w, proj_b)
    return out


def _sine_pos_enc(coords, dim):
    freqs = jnp.arange(dim // 4, dtype=jnp.float32)
    wx = 3.14 * coords[..., 0:1] * freqs * (1.0 / 200.0)
    wy = 3.14 * coords[..., 1:2] * freqs * (1.0 / 200.0)
    return jnp.concatenate([jnp.sin(wx), jnp.cos(wx), jnp.sin(wy), jnp.cos(wy)],
                           axis=-1)


def kernel(x, context, ctx_w, ctx_b, q_w, q_b, k_w, k_b, v_w, v_b,
           proj_w, proj_b):
    B, N, C = x.shape
    H = W = int(math.isqrt(N))
    ws = 7
    nh = 8
    vert_c_dim = q_w.shape[0] - C
    C_qk = C + vert_c_dim
    hd = C // nh
    scale = hd ** (-0.5)
    bn = B // 5
    _h, nwx = H // ws, W // ws
    planep = _h * nwx * ws * 8

    q_w = q_w * scale
    q_b = q_b * scale

    wqk = jnp.concatenate([q_w, k_w], axis=1)
    w_ctx2qk = ctx_w @ wqk[C:]
    bias = (jnp.concatenate([q_b, k_b], axis=0) + ctx_b @ wqk[C:]).reshape(1, 2 * C)

    rr, wxs, yy, xx = jnp.meshgrid(
        jnp.arange(_h, dtype=jnp.float32), jnp.arange(nwx, dtype=jnp.float32),
        jnp.arange(ws, dtype=jnp.float32), jnp.arange(8, dtype=jnp.float32),
        indexing="ij")
    cx = (wxs * ws + jnp.minimum(xx, ws - 1.0)).reshape(-1)
    cy = (rr * ws + yy).reshape(-1)
    coords_wm = jnp.stack([cx, cy], axis=-1)
    ce_wm = _sine_pos_enc(coords_wm, C_qk)

    ctx_raw = context.reshape(context.shape[1], H * W)

    add_q, add_k = _prep_terms(ce_wm, ctx_raw, wqk, w_ctx2qk, bias,
                               C=C, _h=_h, nwx=nwx, ws=ws)

    x6 = x.reshape(B, _h, ws, nwx, ws, C)
    w_kv = jnp.concatenate([k_w[:C], v_w], axis=1)

    out = _fused_attention(x6, add_q, add_k, q_w[:C], w_kv,
                           v_b.reshape(1, C), proj_w, proj_b.reshape(1, C),
                           ws=ws, nwx=nwx, nh=nh, bn=bn)
    return out.reshape(B, N, C)


# speedup vs baseline: 2.7312x; 1.0355x over previous
import functools
import math

import jax
import jax.numpy as jnp
from jax import lax
from jax.experimental import pallas as pl
from jax.experimental.pallas import tpu as pltpu


def _prep_kernel(ce_ref, ctx_ref, ctxw_ref, qw_ref, kw_ref, vw_ref, qb_ref,
                 kb_ref, cb_ref, aq_ref, ak_ref, wqo_ref, wkvo_ref,
                 *, C, _h, nwx, ws, scale):
    wqk = jnp.concatenate([qw_ref[...] * scale, kw_ref[...]], axis=1)
    wqo_ref[...] = wqk[:C, :C]
    wkvo_ref[...] = jnp.concatenate([wqk[:C, C:], vw_ref[...]], axis=1)
    w_ctx2qk = jnp.dot(ctxw_ref[...], wqk[C:, :],
                       preferred_element_type=jnp.float32)
    bias = (jnp.concatenate([qb_ref[...] * scale, kb_ref[...]], axis=1)
            + jnp.dot(cb_ref[...], wqk[C:, :],
                      preferred_element_type=jnp.float32))
    y = jnp.dot(ce_ref[...], wqk, preferred_element_type=jnp.float32)
    y2 = lax.dot_general(ctx_ref[...], w_ctx2qk,
                         (((0,), (0,)), ((), ())),
                         preferred_element_type=jnp.float32)
    y2 = y2.reshape(_h, ws, nwx, ws, 2 * C).transpose(0, 2, 1, 3, 4)
    y2 = jnp.pad(y2, ((0, 0), (0, 0), (0, 0), (0, 8 - ws), (0, 0)))
    y2 = y2.reshape(_h * nwx * ws * 8, 2 * C)
    y = y + y2 + bias
    aq_ref[...] = y[:, :C]
    ak_ref[...] = y[:, C:]


def _prep_terms(ce_wm, ctx_raw, ctx_w, q_w, k_w, v_w, q_b, k_b, ctx_b,
                *, C, _h, nwx, ws, scale):
    planep = ce_wm.shape[0]
    C_qk = ce_wm.shape[1]
    full = lambda a: pl.BlockSpec(a.shape, lambda i: tuple(0 for _ in a.shape))
    return pl.pallas_call(
        functools.partial(_prep_kernel, C=C, _h=_h, nwx=nwx, ws=ws,
                          scale=scale),
        out_shape=(jax.ShapeDtypeStruct((planep, C), jnp.float32),
                   jax.ShapeDtypeStruct((planep, C), jnp.float32),
                   jax.ShapeDtypeStruct((C, C), jnp.float32),
                   jax.ShapeDtypeStruct((C, 2 * C), jnp.float32)),
        grid_spec=pltpu.PrefetchScalarGridSpec(
            num_scalar_prefetch=0,
            grid=(1,),
            in_specs=[full(ce_wm), full(ctx_raw), full(ctx_w), full(q_w),
                      full(k_w), full(v_w), full(q_b), full(k_b), full(ctx_b)],
            out_specs=[pl.BlockSpec((planep, C), lambda i: (0, 0)),
                       pl.BlockSpec((planep, C), lambda i: (0, 0)),
                       pl.BlockSpec((C, C), lambda i: (0, 0)),
                       pl.BlockSpec((C, 2 * C), lambda i: (0, 0))],
        ),
        compiler_params=pltpu.CompilerParams(dimension_semantics=("arbitrary",)),
    )(ce_wm, ctx_raw, ctx_w, q_w, k_w, v_w, q_b, k_b, ctx_b)


def _attn_kernel(x_ref, aq_ref, ak_ref, wq_ref, wkv_ref,
                 vb_ref, pw_ref, pb_ref, o_ref, k_ref, v_ref,
                 *, ws, nwx, _h, C, nh, bn):
    b = pl.program_id(0)
    g = b // bn
    hd = C // nh
    Lp = ws * 8
    rows = _h * nwx * Lp

    a = x_ref[0]
    a = jnp.transpose(a, (0, 2, 1, 3, 4))
    a = jnp.pad(a, ((0, 0), (0, 0), (0, 0), (0, 8 - ws), (0, 0)))
    xw = a.reshape(rows, C)

    q = jnp.dot(xw, wq_ref[...], preferred_element_type=jnp.float32)
    q = q + aq_ref[...]
    kv = jnp.dot(xw, wkv_ref[...], preferred_element_type=jnp.float32)
    k_ref[...] = kv[:, :C] + ak_ref[...]
    v_ref[...] = kv[:, C:] + vb_ref[...]

    rid = lax.broadcasted_iota(jnp.int32, (nh * Lp, C), 0) // Lp
    cid = lax.broadcasted_iota(jnp.int32, (nh * Lp, C), 1) // hd
    mask = rid == cid

    lane = lax.broadcasted_iota(jnp.int32, (1, Lp), 1)
    bound = jnp.where(lane % 8 < ws, 60.0, -1e30)

    outs = []
    for r in range(_h):
        ru = min(r + 1, _h - 1)
        rd = max(r - 1, 0)
        kvr = jnp.where(g == 0, ru, jnp.where(g == 1, rd, r))
        for wx in range(nwx):
            wl = min(wx + 1, nwx - 1)
            wr = max(wx - 1, 0)
            kvw = jnp.where(g == 2, wl, jnp.where(g == 3, wr, wx))
            slab = (kvr * nwx + kvw) * Lp
            qw = q[(r * nwx + wx) * Lp:(r * nwx + wx + 1) * Lp]
            kw = k_ref[pl.ds(slab, Lp), :]
            vw = v_ref[pl.ds(slab, Lp), :]
            qrep = jnp.where(mask, jnp.tile(qw, (nh, 1)), 0.0)
            s = lax.dot_general(qrep, kw, (((1,), (1,)), ((), ())),
                                preferred_element_type=jnp.float32)
            p = jnp.exp(jnp.minimum(s, bound))
            p = p * pl.reciprocal(jnp.sum(p, axis=-1, keepdims=True))
            pv = jnp.dot(p, vw, preferred_element_type=jnp.float32)
            pv = jnp.where(mask, pv, 0.0)
            outs.append(pv.reshape(nh, Lp, C).sum(axis=0))

    o_all = jnp.concatenate(outs, axis=0)
    res = jnp.dot(o_all, pw_ref[...], preferred_element_type=jnp.float32)
    res = res + pb_ref[...]
    res = res.reshape(_h, nwx, ws, 8, C)[:, :, :, :ws, :]
    o_ref[0] = jnp.transpose(res, (0, 2, 1, 3, 4))


def _fused_attention(x6, add_q, add_k, w_q, w_kv, v_b, proj_w, proj_b,
                     *, ws, nwx, nh, bn):
    B, _h = x6.shape[0], x6.shape[1]
    C = x6.shape[5]
    rows = _h * nwx * ws * 8

    b_map = lambda b: (b, 0, 0, 0, 0, 0)
    res_map = lambda b: (0, 0)

    out = pl.pallas_call(
        functools.partial(_attn_kernel, ws=ws, nwx=nwx, _h=_h, C=C, nh=nh,
                          bn=bn),
        out_shape=jax.ShapeDtypeStruct((B, _h, ws, nwx, ws, C), jnp.float32),
        grid_spec=pltpu.PrefetchScalarGridSpec(
            num_scalar_prefetch=0,
            grid=(B,),
            in_specs=[
                pl.BlockSpec((1, _h, ws, nwx, ws, C), b_map),
                pl.BlockSpec((rows, C), res_map),
                pl.BlockSpec((rows, C), res_map),
                pl.BlockSpec((C, C), res_map),
                pl.BlockSpec((C, 2 * C), res_map),
                pl.BlockSpec((1, C), res_map),
                pl.BlockSpec((C, C), res_map),
                pl.BlockSpec((1, C), res_map),
            ],
            out_specs=pl.BlockSpec((1, _h, ws, nwx, ws, C), b_map),
            scratch_shapes=[pltpu.VMEM((rows, C), jnp.float32),
                            pltpu.VMEM((rows, C), jnp.float32)],
        ),
        compiler_params=pltpu.CompilerParams(
            dimension_semantics=("parallel",)),
    )(x6, add_q, add_k, w_q, w_kv, v_b, proj_w, proj_b)
    return out


def _sine_pos_enc(coords, dim):
    freqs = jnp.arange(dim // 4, dtype=jnp.float32)
    wx = 3.14 * coords[..., 0:1] * freqs * (1.0 / 200.0)
    wy = 3.14 * coords[..., 1:2] * freqs * (1.0 / 200.0)
    return jnp.concatenate([jnp.sin(wx), jnp.cos(wx), jnp.sin(wy), jnp.cos(wy)],
                           axis=-1)


def kernel(x, context, ctx_w, ctx_b, q_w, q_b, k_w, k_b, v_w, v_b,
           proj_w, proj_b):
    B, N, C = x.shape
    H = W = int(math.isqrt(N))
    ws = 7
    nh = 8
    vert_c_dim = q_w.shape[0] - C
    C_qk = C + vert_c_dim
    hd = C // nh
    scale = hd ** (-0.5)
    bn = B // 5
    _h, nwx = H // ws, W // ws
    planep = _h * nwx * ws * 8

    rr, wxs, yy, xx = jnp.meshgrid(
        jnp.arange(_h, dtype=jnp.float32), jnp.arange(nwx, dtype=jnp.float32),
        jnp.arange(ws, dtype=jnp.float32), jnp.arange(8, dtype=jnp.float32),
        indexing="ij")
    cx = (wxs * ws + jnp.minimum(xx, ws - 1.0)).reshape(-1)
    cy = (rr * ws + yy).reshape(-1)
    coords_wm = jnp.stack([cx, cy], axis=-1)
    ce_wm = _sine_pos_enc(coords_wm, C_qk)

    ctx_raw = context.reshape(context.shape[1], H * W)

    add_q, add_k, wq_s, w_kv = _prep_terms(
        ce_wm, ctx_raw, ctx_w, q_w, k_w, v_w,
        q_b.reshape(1, C), k_b.reshape(1, C), ctx_b.reshape(1, -1),
        C=C, _h=_h, nwx=nwx, ws=ws, scale=scale)

    x6 = x.reshape(B, _h, ws, nwx, ws, C)

    out = _fused_attention(x6, add_q, add_k, wq_s, w_kv,
                           v_b.reshape(1, C), proj_w, proj_b.reshape(1, C),
                           ws=ws, nwx=nwx, nh=nh, bn=bn)
    return out.reshape(B, N, C)
